# baseline (speedup 1.0000x reference)
import functools

import jax
import jax.numpy as jnp
from jax import lax
from jax.experimental import pallas as pl
from jax.experimental.pallas import tpu as pltpu
from jax.experimental.pallas import tpu_sc as plsc

_VOCAB = 1000
_EMBED_DIM = 16
_NUM_WORKERS = 32
_CHUNK = 128
_B = 13
_F = 7


@functools.lru_cache(maxsize=None)
def _make_gather(total: int):
    per_w = total // _NUM_WORKERS
    n_chunks = per_w // _CHUNK
    assert n_chunks % _B == 0
    mesh = plsc.VectorSubcoreMesh(core_axis_name="c", subcore_axis_name="s")

    @functools.partial(
        pl.kernel,
        mesh=mesh,
        out_type=jax.ShapeDtypeStruct((total, _EMBED_DIM), jnp.float32),
        compiler_params=pltpu.CompilerParams(use_tc_tiling_on_sc=False),
        scratch_types=[
            pltpu.VMEM((n_chunks, _CHUNK), jnp.int32),
            pltpu.VMEM((_B, _CHUNK, _EMBED_DIM), jnp.float32),
        ]
        + [pltpu.SemaphoreType.DMA] * (2 * _B),
    )
    def gather_kernel(table_hbm, idx_hbm, out_hbm, idx_v, rows_v, *sems):
        sem_g = sems[:_B]
        sem_w = sems[_B:]
        wid = lax.axis_index("s") * 2 + lax.axis_index("c")
        pltpu.sync_copy(idx_hbm.at[wid], idx_v)
        base = wid * per_w

        def wait_gather(b):
            pltpu.make_async_copy(
                out_hbm.at[pl.ds(0, _CHUNK)], rows_v.at[b], sem_g[b]
            ).wait()

        def wait_write(b):
            pltpu.make_async_copy(
                rows_v.at[b], out_hbm.at[pl.ds(0, _CHUNK)], sem_w[b]
            ).wait()

        for b in range(_F):
            pltpu.async_copy(table_hbm.at[idx_v.at[b]], rows_v.at[b], sem_g[b])

        def body(g, carry):
            j0 = g * _B
            for b in range(_B):
                j = j0 + b
                jn = j + _F
                bn = (b + _F) % _B
                wait_gather(b)
                pltpu.async_copy(
                    rows_v.at[b],
                    out_hbm.at[pl.ds(base + j * _CHUNK, _CHUNK)],
                    sem_w[b],
                )

                @pl.when(jn < n_chunks)
                def _():
                    @pl.when(jn - _B >= 0)
                    def _():
                        wait_write(bn)

                    pltpu.async_copy(
                        table_hbm.at[idx_v.at[jn]], rows_v.at[bn], sem_g[bn]
                    )

            return carry

        lax.fori_loop(0, n_chunks // _B, body, 0)
        for b in range(_B):
            wait_write(b)

    return gather_kernel


def kernel(tokens, tables):
    b, w, nf = tokens.shape
    total = b * w * nf
    flat_table = tables.reshape(nf * _VOCAB, _EMBED_DIM)
    offs = jnp.arange(nf, dtype=jnp.int32) * _VOCAB
    flat_idx = (tokens.astype(jnp.int32) + offs).reshape(
        _NUM_WORKERS, total // _NUM_WORKERS // _CHUNK, _CHUNK
    )
    out = _make_gather(total)(flat_table, flat_idx)
    return out.reshape(b, w, nf * _EMBED_DIM)


# speedup vs baseline: 1.0257x; 1.0257x over previous
import functools

import jax
import jax.numpy as jnp
from jax import lax
from jax.experimental import pallas as pl
from jax.experimental.pallas import tpu as pltpu
from jax.experimental.pallas import tpu_sc as plsc

_VOCAB = 1000
_EMBED_DIM = 16
_NUM_WORKERS = 32
_CHUNK = 128
_B = 13
_F = 7
_N_SP = 338
_N_HBM = 312


@functools.lru_cache(maxsize=None)
def _make_gather(total: int):
    n_chunks = total // _CHUNK
    assert 16 * (_N_SP + _N_HBM) == n_chunks
    assert _N_SP % _B == 0 and _N_HBM % _B == 0
    mesh = plsc.VectorSubcoreMesh(core_axis_name="c", subcore_axis_name="s")

    @functools.partial(
        pl.kernel,
        mesh=mesh,
        out_type=jax.ShapeDtypeStruct((total, _EMBED_DIM), jnp.float32),
        compiler_params=pltpu.CompilerParams(use_tc_tiling_on_sc=False),
        scratch_types=[
            pltpu.VMEM((_N_SP, _CHUNK), jnp.int32),
            pltpu.VMEM((_B, _CHUNK, _EMBED_DIM), jnp.float32),
            pltpu.VMEM_SHARED((26 * _VOCAB, _EMBED_DIM), jnp.float32),
        ]
        + [pltpu.SemaphoreType.DMA] * (2 * _B),
    )
    def gather_kernel(table_hbm, idx_hbm, out_hbm, idx_v, rows_v, table_sp, *sems):
        sem_g = sems[:_B]
        sem_w = sems[_B:]
        sid = lax.axis_index("s")
        wid = sid * 2 + lax.axis_index("c")
        stage_rows = (26 * _VOCAB) // 16
        pltpu.sync_copy(
            table_hbm.at[pl.ds(sid * stage_rows, stage_rows)],
            table_sp.at[pl.ds(sid * stage_rows, stage_rows)],
        )

        use_sp = sid % 2 == 0
        count = jnp.where(use_sp, _N_SP, _N_HBM)
        ngroups = count // _B
        q = wid // 4
        r = wid % 4
        start = q * (2 * (_N_SP + _N_HBM)) + jnp.where(
            r < 2, r * _N_SP, 2 * _N_SP + (r - 2) * _N_HBM
        )

        @pl.when(use_sp)
        def _():
            pltpu.sync_copy(idx_hbm.at[pl.ds(start, _N_SP)], idx_v)

        @pl.when(jnp.logical_not(use_sp))
        def _():
            pltpu.sync_copy(
                idx_hbm.at[pl.ds(start, _N_HBM)], idx_v.at[pl.ds(0, _N_HBM)]
            )

        plsc.subcore_barrier()

        def fire_gather(j, b):
            @pl.when(use_sp)
            def _():
                pltpu.async_copy(table_sp.at[idx_v.at[j]], rows_v.at[b], sem_g[b])

            @pl.when(jnp.logical_not(use_sp))
            def _():
                pltpu.async_copy(table_hbm.at[idx_v.at[j]], rows_v.at[b], sem_g[b])

        def wait_gather(b):
            pltpu.make_async_copy(
                out_hbm.at[pl.ds(0, _CHUNK)], rows_v.at[b], sem_g[b]
            ).wait()

        def wait_write(b):
            pltpu.make_async_copy(
                rows_v.at[b], out_hbm.at[pl.ds(0, _CHUNK)], sem_w[b]
            ).wait()

        for b in range(_F):
            fire_gather(b, b)

        def body(g, carry):
            j0 = g * _B
            for b in range(_B):
                j = j0 + b
                jn = j + _F
                bn = (b + _F) % _B
                wait_gather(b)
                pltpu.async_copy(
                    rows_v.at[b],
                    out_hbm.at[pl.ds((start + j) * _CHUNK, _CHUNK)],
                    sem_w[b],
                )

                @pl.when(jn < count)
                def _():
                    @pl.when(jn - _B >= 0)
                    def _():
                        wait_write(bn)

                    fire_gather(jn, bn)

            return carry

        lax.fori_loop(0, ngroups, body, 0)
        for b in range(_B):
            wait_write(b)

    return gather_kernel


def kernel(tokens, tables):
    b, w, nf = tokens.shape
    total = b * w * nf
    flat_table = tables.reshape(nf * _VOCAB, _EMBED_DIM)
    offs = jnp.arange(nf, dtype=jnp.int32) * _VOCAB
    flat_idx = (tokens.astype(jnp.int32) + offs).reshape(total // _CHUNK, _CHUNK)
    out = _make_gather(total)(flat_table, flat_idx)
    return out.reshape(b, w, nf * _EMBED_DIM)


# speedup vs baseline: 1.1201x; 1.0920x over previous
import functools

import jax
import jax.numpy as jnp
from jax import lax
from jax.experimental import pallas as pl
from jax.experimental.pallas import tpu as pltpu
from jax.experimental.pallas import tpu_sc as plsc

_VOCAB = 1000
_EMBED_DIM = 16
_NUM_WORKERS = 32
_CHUNK = 128
_B = 13
_F = 7
_N_SP = 325
_N_HBM = 325


@functools.lru_cache(maxsize=None)
def _make_gather(total: int):
    n_chunks = total // _CHUNK
    assert 16 * (_N_SP + _N_HBM) == n_chunks
    assert _N_SP % _B == 0 and _N_HBM % _B == 0
    mesh = plsc.VectorSubcoreMesh(core_axis_name="c", subcore_axis_name="s")

    @functools.partial(
        pl.kernel,
        mesh=mesh,
        out_type=jax.ShapeDtypeStruct((total, _EMBED_DIM), jnp.float32),
        compiler_params=pltpu.CompilerParams(use_tc_tiling_on_sc=False),
        scratch_types=[
            pltpu.VMEM((_N_SP, _CHUNK), jnp.int32),
            pltpu.VMEM((_B, _CHUNK, _EMBED_DIM), jnp.float32),
            pltpu.VMEM_SHARED((26 * _VOCAB, _EMBED_DIM), jnp.float32),
        ]
        + [pltpu.SemaphoreType.DMA] * (2 * _B),
    )
    def gather_kernel(table_hbm, idx_hbm, out_hbm, idx_v, rows_v, table_sp, *sems):
        sem_g = sems[:_B]
        sem_w = sems[_B:]
        sid = lax.axis_index("s")
        wid = sid * 2 + lax.axis_index("c")
        stage_rows = (26 * _VOCAB) // 16
        pltpu.sync_copy(
            table_hbm.at[pl.ds(sid * stage_rows, stage_rows)],
            table_sp.at[pl.ds(sid * stage_rows, stage_rows)],
        )

        count = _N_SP
        ngroups = count // _B
        start = wid * _N_SP
        pltpu.sync_copy(idx_hbm.at[pl.ds(start, _N_SP)], idx_v)

        plsc.subcore_barrier()

        def fire_gather(j, b):
            pltpu.async_copy(table_sp.at[idx_v.at[j]], rows_v.at[b], sem_g[b])

        def wait_gather(b):
            pltpu.make_async_copy(
                out_hbm.at[pl.ds(0, _CHUNK)], rows_v.at[b], sem_g[b]
            ).wait()

        def wait_write(b):
            pltpu.make_async_copy(
                rows_v.at[b], out_hbm.at[pl.ds(0, _CHUNK)], sem_w[b]
            ).wait()

        for b in range(_F):
            fire_gather(b, b)

        def body(g, carry):
            j0 = g * _B
            for b in range(_B):
                j = j0 + b
                jn = j + _F
                bn = (b + _F) % _B
                wait_gather(b)
                pltpu.async_copy(
                    rows_v.at[b],
                    out_hbm.at[pl.ds((start + j) * _CHUNK, _CHUNK)],
                    sem_w[b],
                )

                @pl.when(jn < count)
                def _():
                    @pl.when(jn - _B >= 0)
                    def _():
                        wait_write(bn)

                    fire_gather(jn, bn)

            return carry

        lax.fori_loop(0, ngroups, body, 0)
        for b in range(_B):
            wait_write(b)

    return gather_kernel


def kernel(tokens, tables):
    b, w, nf = tokens.shape
    total = b * w * nf
    flat_table = tables.reshape(nf * _VOCAB, _EMBED_DIM)
    offs = jnp.arange(nf, dtype=jnp.int32) * _VOCAB
    flat_idx = (tokens.astype(jnp.int32) + offs).reshape(total // _CHUNK, _CHUNK)
    out = _make_gather(total)(flat_table, flat_idx)
    return out.reshape(b, w, nf * _EMBED_DIM)
